# baseline (speedup 1.0000x reference)
import functools

import jax
import jax.numpy as jnp
from jax import lax
from jax.experimental import pallas as pl
from jax.experimental.pallas import tpu as pltpu
from jax.experimental.pallas import tpu_sc as plsc

N = 10000
E = 320000
NC = 2
NS = 16
NW = NC * NS
K = 128
NLOOP = 80
EPW = NLOOP * K
EPAD = EPW * NW
NP = 10112
RPT = NP // NS
DEGP = 16 * 640
DUMMY = N

_MESH = plsc.VectorSubcoreMesh(core_axis_name="c", subcore_axis_name="s")


def _make_deg_kernel():
  @functools.partial(
      pl.kernel,
      out_type=[jax.ShapeDtypeStruct((DEGP,), jnp.float32)] * 2,
      mesh=_MESH,
      scratch_types=[
          pltpu.VMEM((NLOOP, K), jnp.int32),
          pltpu.VMEM((K,), jnp.float32),
          pltpu.VMEM((640,), jnp.float32),
          pltpu.VMEM_SHARED((DEGP,), jnp.float32),
      ],
  )
  def deg_kernel(dst_hbm, out0, out1, dst_all, ones_v, bounce_v, deg_sh):
    cid = lax.axis_index("c")
    sid = lax.axis_index("s")
    w = cid * NS + sid
    r0 = sid * 640
    pltpu.sync_copy(dst_hbm.at[w], dst_all)
    for j in range(640 // 16):
      bounce_v[pl.ds(j * 16, 16)] = jnp.zeros((16,), jnp.float32)
    pltpu.sync_copy(bounce_v, deg_sh.at[pl.ds(r0, 640)])
    for j in range(K // 16):
      ones_v[pl.ds(j * 16, 16)] = jnp.ones((16,), jnp.float32)
    plsc.subcore_barrier()

    def body(i, carry):
      pltpu.sync_copy(ones_v, deg_sh.at[dst_all.at[i]], add=True)
      return carry

    lax.fori_loop(0, NLOOP, body, 0)
    plsc.subcore_barrier()

    pltpu.sync_copy(deg_sh.at[pl.ds(r0, 640)], bounce_v)

    @pl.when(cid == 0)
    def _():
      pltpu.sync_copy(bounce_v, out0.at[pl.ds(r0, 640)])

    @pl.when(cid != 0)
    def _():
      pltpu.sync_copy(bounce_v, out1.at[pl.ds(r0, 640)])

  return deg_kernel


def _make_edge_kernel(D, untiled=False):
  params = pltpu.CompilerParams(use_tc_tiling_on_sc=False) if untiled else None

  @functools.partial(
      pl.kernel,
      out_type=[jax.ShapeDtypeStruct((NP, D), jnp.float32)] * 2,
      mesh=_MESH,
      compiler_params=params,
      scratch_types=[
          pltpu.VMEM((NLOOP // 2, K), jnp.int32),
          pltpu.VMEM((NLOOP // 2, K), jnp.int32),
          pltpu.VMEM((K, D), jnp.float32),
          pltpu.VMEM((K, D), jnp.float32),
          pltpu.VMEM_SHARED((NP, D), jnp.float32),
          pltpu.SemaphoreType.DMA,
          pltpu.SemaphoreType.DMA,
          pltpu.SemaphoreType.DMA,
          pltpu.SemaphoreType.DMA,
      ],
  )
  def edge_kernel(src_hbm, dst_hbm, h_hbm, z_hbm, out0, out1,
                  src_all, dst_all, buf_a, buf_b, acc_sh,
                  sem_ga, sem_gb, sem_sa, sem_sb):
    cid = lax.axis_index("c")
    sid = lax.axis_index("s")
    w = cid * NS + sid
    r0 = sid * RPT
    NH = NLOOP // 2
    NG = NH // 2

    @pl.when(cid == 0)
    def _():
      pltpu.sync_copy(h_hbm.at[pl.ds(r0, RPT)], acc_sh.at[pl.ds(r0, RPT)])

    @pl.when(cid != 0)
    def _():
      pltpu.sync_copy(z_hbm.at[pl.ds(r0, RPT)], acc_sh.at[pl.ds(r0, RPT)])

    plsc.subcore_barrier()

    for ph in range(2):
      pltpu.sync_copy(src_hbm.at[w, pl.ds(ph * NH, NH)], src_all)
      pltpu.sync_copy(dst_hbm.at[w, pl.ds(ph * NH, NH)], dst_all)
      pltpu.async_copy(h_hbm.at[src_all.at[0]], buf_a, sem_ga)
      pltpu.async_copy(h_hbm.at[src_all.at[1]], buf_b, sem_gb)

      def body(g, carry):
        i0 = 2 * g
        i1 = i0 + 1
        pltpu.make_async_copy(h_hbm.at[pl.ds(0, K)], buf_a, sem_ga).wait()
        pltpu.async_copy(buf_a, acc_sh.at[dst_all.at[i0]], sem_sa, add=True)
        pltpu.make_async_copy(h_hbm.at[pl.ds(0, K)], buf_b, sem_gb).wait()
        pltpu.async_copy(buf_b, acc_sh.at[dst_all.at[i1]], sem_sb, add=True)

        @pl.when(g < NG - 1)
        def _():
          pltpu.make_async_copy(h_hbm.at[pl.ds(0, K)], buf_a, sem_sa).wait()
          pltpu.async_copy(h_hbm.at[src_all.at[i0 + 2]], buf_a, sem_ga)
          pltpu.make_async_copy(h_hbm.at[pl.ds(0, K)], buf_b, sem_sb).wait()
          pltpu.async_copy(h_hbm.at[src_all.at[i1 + 2]], buf_b, sem_gb)
        return carry

      lax.fori_loop(0, NG, body, 0)
      pltpu.make_async_copy(h_hbm.at[pl.ds(0, K)], buf_a, sem_sa).wait()
      pltpu.make_async_copy(h_hbm.at[pl.ds(0, K)], buf_b, sem_sb).wait()
    plsc.subcore_barrier()

    @pl.when(cid == 0)
    def _():
      pltpu.sync_copy(acc_sh.at[pl.ds(r0, RPT)], out0.at[pl.ds(r0, RPT)])

    @pl.when(cid != 0)
    def _():
      pltpu.sync_copy(acc_sh.at[pl.ds(r0, RPT)], out1.at[pl.ds(r0, RPT)])

  return edge_kernel


_deg_call = _make_deg_kernel()
_edge128 = _make_edge_kernel(128)
_edge64 = _make_edge_kernel(64, untiled=True)


RB = 512
GRID = (N + RB - 1) // RB


def _tc_scale_matmul(x, W, deg0, deg1):
  D_in, D_out = W.shape

  def body(x_ref, w_ref, d0_ref, d1_ref, o_ref):
    dinv = lax.rsqrt(d0_ref[...] + d1_ref[...] + 1.0)
    h = jnp.dot(x_ref[...], w_ref[...], preferred_element_type=jnp.float32)
    o_ref[...] = h * dinv[:, None]

  return pl.pallas_call(
      body,
      grid=(GRID,),
      in_specs=[
          pl.BlockSpec((RB, D_in), lambda i: (i, 0)),
          pl.BlockSpec((D_in, D_out), lambda i: (0, 0)),
          pl.BlockSpec((RB,), lambda i: (i,)),
          pl.BlockSpec((RB,), lambda i: (i,)),
      ],
      out_specs=pl.BlockSpec((RB, D_out), lambda i: (i, 0)),
      out_shape=jax.ShapeDtypeStruct((NP, D_out), jnp.float32),
  )(x, W, deg0, deg1)


def _tc_mid(a0, a1, deg0, deg1, b1, W2):
  D_in, D_out = W2.shape

  def body(a0_ref, a1_ref, d0_ref, d1_ref, b_ref, w_ref, o_ref):
    dinv = lax.rsqrt(d0_ref[...] + d1_ref[...] + 1.0)
    o1 = jnp.maximum((a0_ref[...] + a1_ref[...]) * dinv[:, None] + b_ref[...],
                     0.0)
    h2 = jnp.dot(o1, w_ref[...], preferred_element_type=jnp.float32)
    o_ref[...] = h2 * dinv[:, None]

  return pl.pallas_call(
      body,
      grid=(GRID,),
      in_specs=[
          pl.BlockSpec((RB, D_in), lambda i: (i, 0)),
          pl.BlockSpec((RB, D_in), lambda i: (i, 0)),
          pl.BlockSpec((RB,), lambda i: (i,)),
          pl.BlockSpec((RB,), lambda i: (i,)),
          pl.BlockSpec((1, D_in), lambda i: (0, 0)),
          pl.BlockSpec((D_in, D_out), lambda i: (0, 0)),
      ],
      out_specs=pl.BlockSpec((RB, D_out), lambda i: (i, 0)),
      out_shape=jax.ShapeDtypeStruct((NP, D_out), jnp.float32),
  )(a0, a1, deg0, deg1, b1, W2)


def _tc_final(a0, a1, deg0, deg1, b2):
  D = b2.shape[1]

  def body(a0_ref, a1_ref, d0_ref, d1_ref, b_ref, o_ref):
    dinv = lax.rsqrt(d0_ref[...] + d1_ref[...] + 1.0)
    o_ref[...] = (a0_ref[...] + a1_ref[...]) * dinv[:, None] + b_ref[...]

  return pl.pallas_call(
      body,
      grid=(GRID,),
      in_specs=[
          pl.BlockSpec((RB, D), lambda i: (i, 0)),
          pl.BlockSpec((RB, D), lambda i: (i, 0)),
          pl.BlockSpec((RB,), lambda i: (i,)),
          pl.BlockSpec((RB,), lambda i: (i,)),
          pl.BlockSpec((1, D), lambda i: (0, 0)),
      ],
      out_specs=pl.BlockSpec((RB, D), lambda i: (i, 0)),
      out_shape=jax.ShapeDtypeStruct((N, D), jnp.float32),
  )(a0, a1, deg0, deg1, b2)


@jax.jit
def kernel(x, edge_index, W1, b1, W2, b2):
  src = edge_index[0].astype(jnp.int32)
  dst = edge_index[1].astype(jnp.int32)
  npad = EPAD - E
  pad_i = jnp.arange(npad, dtype=jnp.int32)
  src_p = jnp.concatenate([src, pad_i % N]).reshape(NW, NLOOP, K)
  dst_p = jnp.concatenate(
      [dst, DUMMY + pad_i % (NP - N)]).reshape(NW, NLOOP, K)
  z128 = jnp.zeros((NP, 128), jnp.float32)
  z64 = jnp.zeros((NP, 64), jnp.float32)

  deg0, deg1 = _deg_call(dst_p)
  h1s = _tc_scale_matmul(x, W1, deg0, deg1)
  a10, a11 = _edge128(src_p, dst_p, h1s, z128)
  h2s = _tc_mid(a10, a11, deg0, deg1, b1.reshape(1, -1), W2)
  a20, a21 = _edge64(src_p, dst_p, h2s, z64)
  return _tc_final(a20, a21, deg0, deg1, b2.reshape(1, -1))


# speedup vs baseline: 1.1846x; 1.1846x over previous
import functools

import jax
import jax.numpy as jnp
from jax import lax
from jax.experimental import pallas as pl
from jax.experimental.pallas import tpu as pltpu
from jax.experimental.pallas import tpu_sc as plsc

N = 10000
E = 320000
NC = 2
NS = 16
NW = NC * NS
K = 128
NLOOP = 80
EPW = NLOOP * K
EPAD = EPW * NW
NP = 10112
RPT = NP // NS
DEGP = 16 * 640
DUMMY = N

_MESH = plsc.VectorSubcoreMesh(core_axis_name="c", subcore_axis_name="s")


def _make_deg_kernel():
  @functools.partial(
      pl.kernel,
      out_type=[jax.ShapeDtypeStruct((DEGP,), jnp.float32)] * 2,
      mesh=_MESH,
      scratch_types=[
          pltpu.VMEM((NLOOP, K), jnp.int32),
          pltpu.VMEM((K,), jnp.float32),
          pltpu.VMEM((640,), jnp.float32),
          pltpu.VMEM_SHARED((DEGP,), jnp.float32),
      ],
  )
  def deg_kernel(dst_hbm, out0, out1, dst_all, ones_v, bounce_v, deg_sh):
    cid = lax.axis_index("c")
    sid = lax.axis_index("s")
    w = cid * NS + sid
    r0 = sid * 640
    pltpu.sync_copy(dst_hbm.at[w], dst_all)
    for j in range(640 // 16):
      bounce_v[pl.ds(j * 16, 16)] = jnp.zeros((16,), jnp.float32)
    pltpu.sync_copy(bounce_v, deg_sh.at[pl.ds(r0, 640)])
    for j in range(K // 16):
      ones_v[pl.ds(j * 16, 16)] = jnp.ones((16,), jnp.float32)
    plsc.subcore_barrier()

    def body(i, carry):
      pltpu.sync_copy(ones_v, deg_sh.at[dst_all.at[i]], add=True)
      return carry

    lax.fori_loop(0, NLOOP, body, 0)
    plsc.subcore_barrier()

    pltpu.sync_copy(deg_sh.at[pl.ds(r0, 640)], bounce_v)

    @pl.when(cid == 0)
    def _():
      pltpu.sync_copy(bounce_v, out0.at[pl.ds(r0, 640)])

    @pl.when(cid != 0)
    def _():
      pltpu.sync_copy(bounce_v, out1.at[pl.ds(r0, 640)])

  return deg_kernel


def _make_edge_kernel(D, untiled=False):
  params = pltpu.CompilerParams(use_tc_tiling_on_sc=False) if untiled else None

  @functools.partial(
      pl.kernel,
      out_type=[jax.ShapeDtypeStruct((NP, D), jnp.float32)] * 2,
      mesh=_MESH,
      compiler_params=params,
      scratch_types=[
          pltpu.VMEM((NLOOP // 2, K), jnp.int32),
          pltpu.VMEM((NLOOP // 2, K), jnp.int32),
          pltpu.VMEM((K, D), jnp.float32),
          pltpu.VMEM((K, D), jnp.float32),
          pltpu.VMEM_SHARED((NP, D), jnp.float32),
          pltpu.SemaphoreType.DMA,
          pltpu.SemaphoreType.DMA,
      ],
  )
  def edge_kernel(src_hbm, dst_hbm, h_hbm, z_hbm, out0, out1,
                  src_all, dst_all, buf_a, buf_b, acc_sh, sem_a, sem_b):
    cid = lax.axis_index("c")
    sid = lax.axis_index("s")
    w = cid * NS + sid
    r0 = sid * RPT
    NH = NLOOP // 2
    NG = NH // 2

    @pl.when(cid == 0)
    def _():
      pltpu.sync_copy(h_hbm.at[pl.ds(r0, RPT)], acc_sh.at[pl.ds(r0, RPT)])

    @pl.when(cid != 0)
    def _():
      pltpu.sync_copy(z_hbm.at[pl.ds(r0, RPT)], acc_sh.at[pl.ds(r0, RPT)])

    plsc.subcore_barrier()

    for ph in range(2):
      pltpu.sync_copy(src_hbm.at[w, pl.ds(ph * NH, NH)], src_all)
      pltpu.sync_copy(dst_hbm.at[w, pl.ds(ph * NH, NH)], dst_all)
      pltpu.async_copy(h_hbm.at[src_all.at[0]], buf_a, sem_a)

      def body(g, carry):
        i0 = 2 * g
        i1 = i0 + 1
        pltpu.async_copy(h_hbm.at[src_all.at[i1]], buf_b, sem_b)
        pltpu.make_async_copy(h_hbm.at[pl.ds(0, K)], buf_a, sem_a).wait()
        pltpu.sync_copy(buf_a, acc_sh.at[dst_all.at[i0]], add=True)

        @pl.when(g < NG - 1)
        def _():
          pltpu.async_copy(h_hbm.at[src_all.at[i0 + 2]], buf_a, sem_a)

        pltpu.make_async_copy(h_hbm.at[pl.ds(0, K)], buf_b, sem_b).wait()
        pltpu.sync_copy(buf_b, acc_sh.at[dst_all.at[i1]], add=True)
        return carry

      lax.fori_loop(0, NG, body, 0)
    plsc.subcore_barrier()

    @pl.when(cid == 0)
    def _():
      pltpu.sync_copy(acc_sh.at[pl.ds(r0, RPT)], out0.at[pl.ds(r0, RPT)])

    @pl.when(cid != 0)
    def _():
      pltpu.sync_copy(acc_sh.at[pl.ds(r0, RPT)], out1.at[pl.ds(r0, RPT)])

  return edge_kernel


_deg_call = _make_deg_kernel()
_edge128 = _make_edge_kernel(128)
_edge64 = _make_edge_kernel(64, untiled=True)


RB = 512
GRID = (N + RB - 1) // RB


def _tc_scale_matmul(x, W, deg0, deg1):
  D_in, D_out = W.shape

  def body(x_ref, w_ref, d0_ref, d1_ref, o_ref):
    dinv = lax.rsqrt(d0_ref[...] + d1_ref[...] + 1.0)
    h = jnp.dot(x_ref[...], w_ref[...], preferred_element_type=jnp.float32)
    o_ref[...] = h * dinv[:, None]

  return pl.pallas_call(
      body,
      grid=(GRID,),
      in_specs=[
          pl.BlockSpec((RB, D_in), lambda i: (i, 0)),
          pl.BlockSpec((D_in, D_out), lambda i: (0, 0)),
          pl.BlockSpec((RB,), lambda i: (i,)),
          pl.BlockSpec((RB,), lambda i: (i,)),
      ],
      out_specs=pl.BlockSpec((RB, D_out), lambda i: (i, 0)),
      out_shape=jax.ShapeDtypeStruct((NP, D_out), jnp.float32),
  )(x, W, deg0, deg1)


def _tc_mid(a0, a1, deg0, deg1, b1, W2):
  D_in, D_out = W2.shape

  def body(a0_ref, a1_ref, d0_ref, d1_ref, b_ref, w_ref, o_ref):
    dinv = lax.rsqrt(d0_ref[...] + d1_ref[...] + 1.0)
    o1 = jnp.maximum((a0_ref[...] + a1_ref[...]) * dinv[:, None] + b_ref[...],
                     0.0)
    h2 = jnp.dot(o1, w_ref[...], preferred_element_type=jnp.float32)
    o_ref[...] = h2 * dinv[:, None]

  return pl.pallas_call(
      body,
      grid=(GRID,),
      in_specs=[
          pl.BlockSpec((RB, D_in), lambda i: (i, 0)),
          pl.BlockSpec((RB, D_in), lambda i: (i, 0)),
          pl.BlockSpec((RB,), lambda i: (i,)),
          pl.BlockSpec((RB,), lambda i: (i,)),
          pl.BlockSpec((1, D_in), lambda i: (0, 0)),
          pl.BlockSpec((D_in, D_out), lambda i: (0, 0)),
      ],
      out_specs=pl.BlockSpec((RB, D_out), lambda i: (i, 0)),
      out_shape=jax.ShapeDtypeStruct((NP, D_out), jnp.float32),
  )(a0, a1, deg0, deg1, b1, W2)


def _tc_final(a0, a1, deg0, deg1, b2):
  D = b2.shape[1]

  def body(a0_ref, a1_ref, d0_ref, d1_ref, b_ref, o_ref):
    dinv = lax.rsqrt(d0_ref[...] + d1_ref[...] + 1.0)
    o_ref[...] = (a0_ref[...] + a1_ref[...]) * dinv[:, None] + b_ref[...]

  return pl.pallas_call(
      body,
      grid=(GRID,),
      in_specs=[
          pl.BlockSpec((RB, D), lambda i: (i, 0)),
          pl.BlockSpec((RB, D), lambda i: (i, 0)),
          pl.BlockSpec((RB,), lambda i: (i,)),
          pl.BlockSpec((RB,), lambda i: (i,)),
          pl.BlockSpec((1, D), lambda i: (0, 0)),
      ],
      out_specs=pl.BlockSpec((RB, D), lambda i: (i, 0)),
      out_shape=jax.ShapeDtypeStruct((N, D), jnp.float32),
  )(a0, a1, deg0, deg1, b2)


@jax.jit
def kernel(x, edge_index, W1, b1, W2, b2):
  src = edge_index[0].astype(jnp.int32)
  dst = edge_index[1].astype(jnp.int32)
  npad = EPAD - E
  pad_i = jnp.arange(npad, dtype=jnp.int32)
  src_p = jnp.concatenate([src, pad_i % N]).reshape(NW, NLOOP, K)
  dst_p = jnp.concatenate(
      [dst, DUMMY + pad_i % (NP - N)]).reshape(NW, NLOOP, K)
  z128 = jnp.zeros((NP, 128), jnp.float32)
  z64 = jnp.zeros((NP, 64), jnp.float32)

  deg0, deg1 = _deg_call(dst_p)
  h1s = _tc_scale_matmul(x, W1, deg0, deg1)
  a10, a11 = _edge128(src_p, dst_p, h1s, z128)
  h2s = _tc_mid(a10, a11, deg0, deg1, b1.reshape(1, -1), W2)
  a20, a21 = _edge64(src_p, dst_p, h2s, z64)
  return _tc_final(a20, a21, deg0, deg1, b2.reshape(1, -1))


# speedup vs baseline: 1.2097x; 1.0212x over previous
import functools

import jax
import jax.numpy as jnp
from jax import lax
from jax.experimental import pallas as pl
from jax.experimental.pallas import tpu as pltpu
from jax.experimental.pallas import tpu_sc as plsc

N = 10000
E = 320000
NC = 2
NS = 16
NW = NC * NS
K = 128
NLOOP = 80
EPW = NLOOP * K
EPAD = EPW * NW
NP = 10112
RPT = NP // NS
DEGP = 16 * 640
DUMMY = N

_MESH = plsc.VectorSubcoreMesh(core_axis_name="c", subcore_axis_name="s")


def _make_deg_kernel():
  @functools.partial(
      pl.kernel,
      out_type=[jax.ShapeDtypeStruct((DEGP,), jnp.float32)] * 2,
      mesh=_MESH,
      scratch_types=[
          pltpu.VMEM((NLOOP, K), jnp.int32),
          pltpu.VMEM((K,), jnp.float32),
          pltpu.VMEM((640,), jnp.float32),
          pltpu.VMEM_SHARED((DEGP,), jnp.float32),
      ],
  )
  def deg_kernel(dst_hbm, out0, out1, dst_all, ones_v, bounce_v, deg_sh):
    cid = lax.axis_index("c")
    sid = lax.axis_index("s")
    w = cid * NS + sid
    r0 = sid * 640
    pltpu.sync_copy(dst_hbm.at[w], dst_all)
    for j in range(640 // 16):
      bounce_v[pl.ds(j * 16, 16)] = jnp.zeros((16,), jnp.float32)
    pltpu.sync_copy(bounce_v, deg_sh.at[pl.ds(r0, 640)])
    for j in range(K // 16):
      ones_v[pl.ds(j * 16, 16)] = jnp.ones((16,), jnp.float32)
    plsc.subcore_barrier()

    def body(i, carry):
      pltpu.sync_copy(ones_v, deg_sh.at[dst_all.at[i]], add=True)
      return carry

    lax.fori_loop(0, NLOOP, body, 0)
    plsc.subcore_barrier()

    pltpu.sync_copy(deg_sh.at[pl.ds(r0, 640)], bounce_v)

    @pl.when(cid == 0)
    def _():
      pltpu.sync_copy(bounce_v, out0.at[pl.ds(r0, 640)])

    @pl.when(cid != 0)
    def _():
      pltpu.sync_copy(bounce_v, out1.at[pl.ds(r0, 640)])

  return deg_kernel


def _make_edge_kernel(D, untiled=False, nphase=2):
  params = pltpu.CompilerParams(use_tc_tiling_on_sc=False) if untiled else None
  NH = NLOOP // nphase
  NG = NH // 2

  @functools.partial(
      pl.kernel,
      out_type=[jax.ShapeDtypeStruct((NP, D), jnp.float32)] * 2,
      mesh=_MESH,
      compiler_params=params,
      scratch_types=[
          pltpu.VMEM((NH, K), jnp.int32),
          pltpu.VMEM((NH, K), jnp.int32),
          pltpu.VMEM((K, D), jnp.float32),
          pltpu.VMEM((K, D), jnp.float32),
          pltpu.VMEM_SHARED((NP, D), jnp.float32),
          pltpu.SemaphoreType.DMA,
          pltpu.SemaphoreType.DMA,
      ],
  )
  def edge_kernel(src_hbm, dst_hbm, h_hbm, z_hbm, out0, out1,
                  src_all, dst_all, buf_a, buf_b, acc_sh, sem_a, sem_b):
    cid = lax.axis_index("c")
    sid = lax.axis_index("s")
    w = cid * NS + sid
    r0 = sid * RPT

    pltpu.async_copy(src_hbm.at[w, pl.ds(0, NH)], src_all, sem_a)
    pltpu.async_copy(dst_hbm.at[w, pl.ds(0, NH)], dst_all, sem_b)

    @pl.when(cid == 0)
    def _():
      pltpu.sync_copy(h_hbm.at[pl.ds(r0, RPT)], acc_sh.at[pl.ds(r0, RPT)])

    @pl.when(cid != 0)
    def _():
      pltpu.sync_copy(z_hbm.at[pl.ds(r0, RPT)], acc_sh.at[pl.ds(r0, RPT)])

    pltpu.make_async_copy(src_hbm.at[w, pl.ds(0, NH)], src_all, sem_a).wait()
    pltpu.make_async_copy(dst_hbm.at[w, pl.ds(0, NH)], dst_all, sem_b).wait()
    plsc.subcore_barrier()

    for ph in range(nphase):
      if ph > 0:
        pltpu.sync_copy(src_hbm.at[w, pl.ds(ph * NH, NH)], src_all)
        pltpu.sync_copy(dst_hbm.at[w, pl.ds(ph * NH, NH)], dst_all)
      pltpu.async_copy(h_hbm.at[src_all.at[0]], buf_a, sem_a)

      def body(g, carry):
        i0 = 2 * g
        i1 = i0 + 1
        pltpu.async_copy(h_hbm.at[src_all.at[i1]], buf_b, sem_b)
        pltpu.make_async_copy(h_hbm.at[pl.ds(0, K)], buf_a, sem_a).wait()
        pltpu.sync_copy(buf_a, acc_sh.at[dst_all.at[i0]], add=True)

        @pl.when(g < NG - 1)
        def _():
          pltpu.async_copy(h_hbm.at[src_all.at[i0 + 2]], buf_a, sem_a)

        pltpu.make_async_copy(h_hbm.at[pl.ds(0, K)], buf_b, sem_b).wait()
        pltpu.sync_copy(buf_b, acc_sh.at[dst_all.at[i1]], add=True)
        return carry

      lax.fori_loop(0, NG, body, 0)
    plsc.subcore_barrier()

    @pl.when(cid == 0)
    def _():
      pltpu.sync_copy(acc_sh.at[pl.ds(r0, RPT)], out0.at[pl.ds(r0, RPT)])

    @pl.when(cid != 0)
    def _():
      pltpu.sync_copy(acc_sh.at[pl.ds(r0, RPT)], out1.at[pl.ds(r0, RPT)])

  return edge_kernel


_deg_call = _make_deg_kernel()
_edge128 = _make_edge_kernel(128)
_edge64 = _make_edge_kernel(64, untiled=True, nphase=1)


RB = 512
GRID = (N + RB - 1) // RB


def _tc_scale_matmul(x, W, deg0, deg1):
  D_in, D_out = W.shape

  def body(x_ref, w_ref, d0_ref, d1_ref, o_ref):
    dinv = lax.rsqrt(d0_ref[...] + d1_ref[...] + 1.0)
    h = jnp.dot(x_ref[...], w_ref[...], preferred_element_type=jnp.float32)
    o_ref[...] = h * dinv[:, None]

  return pl.pallas_call(
      body,
      grid=(GRID,),
      in_specs=[
          pl.BlockSpec((RB, D_in), lambda i: (i, 0)),
          pl.BlockSpec((D_in, D_out), lambda i: (0, 0)),
          pl.BlockSpec((RB,), lambda i: (i,)),
          pl.BlockSpec((RB,), lambda i: (i,)),
      ],
      out_specs=pl.BlockSpec((RB, D_out), lambda i: (i, 0)),
      out_shape=jax.ShapeDtypeStruct((NP, D_out), jnp.float32),
  )(x, W, deg0, deg1)


def _tc_mid(a0, a1, deg0, deg1, b1, W2):
  D_in, D_out = W2.shape

  def body(a0_ref, a1_ref, d0_ref, d1_ref, b_ref, w_ref, o_ref):
    dinv = lax.rsqrt(d0_ref[...] + d1_ref[...] + 1.0)
    o1 = jnp.maximum((a0_ref[...] + a1_ref[...]) * dinv[:, None] + b_ref[...],
                     0.0)
    h2 = jnp.dot(o1, w_ref[...], preferred_element_type=jnp.float32)
    o_ref[...] = h2 * dinv[:, None]

  return pl.pallas_call(
      body,
      grid=(GRID,),
      in_specs=[
          pl.BlockSpec((RB, D_in), lambda i: (i, 0)),
          pl.BlockSpec((RB, D_in), lambda i: (i, 0)),
          pl.BlockSpec((RB,), lambda i: (i,)),
          pl.BlockSpec((RB,), lambda i: (i,)),
          pl.BlockSpec((1, D_in), lambda i: (0, 0)),
          pl.BlockSpec((D_in, D_out), lambda i: (0, 0)),
      ],
      out_specs=pl.BlockSpec((RB, D_out), lambda i: (i, 0)),
      out_shape=jax.ShapeDtypeStruct((NP, D_out), jnp.float32),
  )(a0, a1, deg0, deg1, b1, W2)


def _tc_final(a0, a1, deg0, deg1, b2):
  D = b2.shape[1]

  def body(a0_ref, a1_ref, d0_ref, d1_ref, b_ref, o_ref):
    dinv = lax.rsqrt(d0_ref[...] + d1_ref[...] + 1.0)
    o_ref[...] = (a0_ref[...] + a1_ref[...]) * dinv[:, None] + b_ref[...]

  return pl.pallas_call(
      body,
      grid=(GRID,),
      in_specs=[
          pl.BlockSpec((RB, D), lambda i: (i, 0)),
          pl.BlockSpec((RB, D), lambda i: (i, 0)),
          pl.BlockSpec((RB,), lambda i: (i,)),
          pl.BlockSpec((RB,), lambda i: (i,)),
          pl.BlockSpec((1, D), lambda i: (0, 0)),
      ],
      out_specs=pl.BlockSpec((RB, D), lambda i: (i, 0)),
      out_shape=jax.ShapeDtypeStruct((N, D), jnp.float32),
  )(a0, a1, deg0, deg1, b2)


@jax.jit
def kernel(x, edge_index, W1, b1, W2, b2):
  src = edge_index[0].astype(jnp.int32)
  dst = edge_index[1].astype(jnp.int32)
  npad = EPAD - E
  pad_i = jnp.arange(npad, dtype=jnp.int32)
  src_p = jnp.concatenate([src, pad_i % N]).reshape(NW, NLOOP, K)
  dst_p = jnp.concatenate(
      [dst, DUMMY + pad_i % (NP - N)]).reshape(NW, NLOOP, K)
  z128 = jnp.zeros((NP, 128), jnp.float32)
  z64 = jnp.zeros((NP, 64), jnp.float32)

  deg0, deg1 = _deg_call(dst_p)
  h1s = _tc_scale_matmul(x, W1, deg0, deg1)
  a10, a11 = _edge128(src_p, dst_p, h1s, z128)
  h2s = _tc_mid(a10, a11, deg0, deg1, b1.reshape(1, -1), W2)
  a20, a21 = _edge64(src_p, dst_p, h2s, z64)
  return _tc_final(a20, a21, deg0, deg1, b2.reshape(1, -1))
